# Initial kernel scaffold; baseline (speedup 1.0000x reference)
import jax
import jax.numpy as jnp
from jax.experimental import pallas as pl


def kernel(output, target):
    raise NotImplementedError("write your pallas kernel here")


# speedup vs baseline: 2.0816x; 2.0816x over previous
import functools

import jax
import jax.numpy as jnp
from jax import lax
from jax.experimental import pallas as pl
from jax.experimental.pallas import tpu as pltpu

_K = 5
_ROWS_PER_STEP = 8


def _body(tgt_ref, x_ref, loss_ref, acc_ref):
    i = pl.program_id(0)
    nsteps = pl.num_programs(0)
    rb, v = x_ref.shape

    @pl.when(i == 0)
    def _init():
        acc_ref[0] = 0.0
        acc_ref[1] = 0.0

    x = x_ref[...]
    tgt = tgt_ref[...]

    col = lax.broadcasted_iota(jnp.int32, (rb, v), 1)
    t = jnp.sum(jnp.where(col == tgt, x, 0.0), axis=1, keepdims=True)

    m = jnp.max(x, axis=1, keepdims=True)
    s = jnp.sum(jnp.exp(x - m), axis=1, keepdims=True)
    lse = m + jnp.log(s)

    above = (x > t) | ((x == t) & (col < tgt))
    cnt = jnp.sum(jnp.where(above, 1.0, 0.0), axis=1, keepdims=True)

    mis = cnt > (_K - 0.5)
    ce = lse - t
    acc_ref[0] += jnp.sum(jnp.where(mis, ce, 0.0))
    acc_ref[1] += jnp.sum(jnp.where(mis, 1.0, 0.0))

    @pl.when(i == nsteps - 1)
    def _fin():
        n = acc_ref[1]
        loss_ref[0, 0] = jnp.where(n > 0.0, acc_ref[0] / jnp.maximum(n, 1.0), 0.0)


def kernel(output, target):
    b, v = output.shape
    grid = b // _ROWS_PER_STEP
    out = pl.pallas_call(
        _body,
        grid=(grid,),
        in_specs=[
            pl.BlockSpec((_ROWS_PER_STEP, 1), lambda i: (i, 0)),
            pl.BlockSpec((_ROWS_PER_STEP, v), lambda i: (i, 0)),
        ],
        out_specs=pl.BlockSpec(memory_space=pltpu.SMEM),
        out_shape=jax.ShapeDtypeStruct((1, 1), jnp.float32),
        scratch_shapes=[pltpu.SMEM((2,), jnp.float32)],
    )(target.reshape(b, 1).astype(jnp.int32), output)
    return out[0, 0]
